# baseline (speedup 1.0000x reference)
import functools

import jax
import jax.numpy as jnp
from jax import lax
from jax.experimental import pallas as pl
from jax.experimental.pallas import tpu as pltpu
from jax.experimental.pallas import tpu_sc as plsc

D_MODEL = 1024
MAX_LEN = 4096
CHAIN_OFFSET = 1000
SEQ_LEN = 4096
BATCH = 4

_R = 32


@functools.lru_cache(maxsize=1)
def _make_sc_gather():
    info = plsc.get_sparse_core_info()
    nc, ns = info.num_cores, info.num_subcores
    nw = nc * ns
    chunk = SEQ_LEN // nw
    nsub = chunk // _R
    mesh = plsc.VectorSubcoreMesh(core_axis_name="c", subcore_axis_name="s")

    @functools.partial(
        pl.kernel,
        mesh=mesh,
        out_type=jax.ShapeDtypeStruct((SEQ_LEN, D_MODEL), jnp.float32),
        scratch_types=[
            pltpu.VMEM((chunk,), jnp.int32),
            pltpu.VMEM((nsub, _R), jnp.int32),
            pltpu.VMEM((_R, D_MODEL), jnp.float32),
            pltpu.VMEM((_R, D_MODEL), jnp.float32),
            pltpu.SemaphoreType.DMA,
            pltpu.SemaphoreType.DMA,
            pltpu.SemaphoreType.DMA,
            pltpu.SemaphoreType.DMA,
        ],
    )
    def _sc_gather(cid_hbm, pos_hbm, out_hbm, cid_v, idx_v, rows0, rows1,
                   gsem0, gsem1, ssem0, ssem1):
        wid = lax.axis_index("s") * nc + lax.axis_index("c")
        base = wid * chunk
        pltpu.sync_copy(cid_hbm.at[pl.ds(base, chunk)], cid_v)
        bufs = (rows0, rows1)
        gsems = (gsem0, gsem1)
        ssems = (ssem0, ssem1)
        conds = []

        def scat_wait(k):
            bb = k % 2

            @pl.when(conds[k])
            def _():
                pltpu.make_async_copy(
                    bufs[bb], out_hbm.at[pl.ds(base + k * _R, _R)],
                    ssems[bb]).wait()

        for i in range(nsub):
            b = i % 2
            adjs = []
            cids = []
            for j in range(_R // 16):
                off = i * _R + j * 16
                cid16 = cid_v[pl.ds(off, 16)]
                pos16 = lax.iota(jnp.int32, 16) + (base + off)
                adj = jnp.clip(pos16 + cid16 * CHAIN_OFFSET, 0, MAX_LEN - 1)
                idx_v[i, pl.ds(j * 16, 16)] = adj
                adjs.append(adj)
                cids.append(cid16)
            cid_a = cids[0][0]
            cid_b = cids[-1][15]
            adj_a = adjs[0][0]
            adj_b = adjs[-1][15]
            is_linear = (cid_a == cid_b) & (adj_b == adj_a + (_R - 1))
            is_const = adj_b == adj_a
            res_mn = adj_a
            conds.append(jnp.logical_not(is_const))

            if i >= 2:
                scat_wait(i - 2)

            @pl.when(is_linear)
            def _():
                start = pl.multiple_of(res_mn, 8)
                pltpu.async_copy(pos_hbm.at[pl.ds(start, _R)], bufs[b],
                                 gsems[b]).wait()
                pltpu.async_copy(bufs[b], out_hbm.at[pl.ds(base + i * _R, _R)],
                                 ssems[b])

            @pl.when(jnp.logical_not(is_linear | is_const))
            def _():
                pltpu.async_copy(pos_hbm.at[idx_v.at[i]], bufs[b],
                                 gsems[b]).wait()
                pltpu.async_copy(bufs[b], out_hbm.at[pl.ds(base + i * _R, _R)],
                                 ssems[b])

        for k in (nsub - 2, nsub - 1):
            scat_wait(k)

    return _sc_gather


_BS = 256
_NB = SEQ_LEN // _BS


def _tc_add_body(src_ref, x_ref, pos_ref, cid_ref, emb_ref, last_ref, o_ref):
    i = pl.program_id(0)
    cid = cid_ref[0, 0, :]
    n_chains = emb_ref.shape[0]
    onehot = (cid[:, None]
              == lax.broadcasted_iota(jnp.int32, (_BS, n_chains), 1)
              ).astype(jnp.float32)
    chain = jnp.dot(onehot, emb_ref[...], preferred_element_type=jnp.float32,
                    precision=lax.Precision.HIGHEST)
    s = lax.broadcasted_iota(jnp.int32, (_BS, 1), 0) + i * _BS
    clamped = (s + cid[:, None] * CHAIN_OFFSET) >= (MAX_LEN - 1)
    pos = jnp.where(clamped, last_ref[...], pos_ref[...])
    enc = pos + chain
    o_ref[...] = x_ref[...] + enc[None, :, :]


def _tc_add(src, x, pos_rows, cid3, chain_embedding, pe_last):
    grid_spec = pltpu.PrefetchScalarGridSpec(
        num_scalar_prefetch=1,
        grid=(_NB,),
        in_specs=[
            pl.BlockSpec((BATCH, _BS, D_MODEL), lambda i, src: (0, i, 0)),
            pl.BlockSpec((_BS, D_MODEL), lambda i, src: (src[i], 0)),
            pl.BlockSpec((1, 1, _BS), lambda i, src: (i, 0, 0)),
            pl.BlockSpec(chain_embedding.shape, lambda i, src: (0, 0)),
            pl.BlockSpec((1, D_MODEL), lambda i, src: (0, 0)),
        ],
        out_specs=pl.BlockSpec((BATCH, _BS, D_MODEL), lambda i, src: (0, i, 0)),
    )
    return pl.pallas_call(
        _tc_add_body,
        grid_spec=grid_spec,
        out_shape=jax.ShapeDtypeStruct(x.shape, x.dtype),
    )(src, x, pos_rows, cid3, chain_embedding, pe_last)


def kernel(x, chain_id_tensor, pos_encoding, chain_embedding):
    cid = chain_id_tensor.astype(jnp.int32)
    pos_rows = _make_sc_gather()(cid, pos_encoding)
    cid3 = cid.reshape(_NB, 1, _BS)
    pe_last = pos_encoding[MAX_LEN - 1:, :]
    blk_ids = jnp.arange(_NB, dtype=jnp.int32)
    first_cid = cid[:: _BS]
    unclamped = (blk_ids * _BS + first_cid * CHAIN_OFFSET) < (MAX_LEN - 1)
    src = lax.cummax(jnp.where(unclamped, blk_ids, 0), axis=0)
    return _tc_add(src, x, pos_rows, cid3, chain_embedding, pe_last)


# speedup vs baseline: 1.0046x; 1.0046x over previous
import functools

import jax
import jax.numpy as jnp
from jax import lax
from jax.experimental import pallas as pl
from jax.experimental.pallas import tpu as pltpu
from jax.experimental.pallas import tpu_sc as plsc

D_MODEL = 1024
MAX_LEN = 4096
CHAIN_OFFSET = 1000
SEQ_LEN = 4096
BATCH = 4

_R = 32
_HALF = SEQ_LEN // 2


@functools.lru_cache(maxsize=2)
def _make_sc_gather(off):
    info = plsc.get_sparse_core_info()
    nc, ns = info.num_cores, info.num_subcores
    nw = nc * ns
    chunk = _HALF // nw
    nsub = chunk // _R
    mesh = plsc.VectorSubcoreMesh(core_axis_name="c", subcore_axis_name="s")

    @functools.partial(
        pl.kernel,
        mesh=mesh,
        out_type=jax.ShapeDtypeStruct((_HALF, D_MODEL), jnp.float32),
        scratch_types=[
            pltpu.VMEM((chunk,), jnp.int32),
            pltpu.VMEM((nsub, _R), jnp.int32),
            pltpu.VMEM((_R, D_MODEL), jnp.float32),
            pltpu.VMEM((_R, D_MODEL), jnp.float32),
            pltpu.SemaphoreType.DMA,
            pltpu.SemaphoreType.DMA,
            pltpu.SemaphoreType.DMA,
            pltpu.SemaphoreType.DMA,
        ],
    )
    def _sc_gather(cid_hbm, pos_hbm, out_hbm, cid_v, idx_v, rows0, rows1,
                   gsem0, gsem1, ssem0, ssem1):
        wid = lax.axis_index("s") * nc + lax.axis_index("c")
        base = wid * chunk
        gbase = off + base
        pltpu.sync_copy(cid_hbm.at[pl.ds(gbase, chunk)], cid_v)
        bufs = (rows0, rows1)
        gsems = (gsem0, gsem1)
        ssems = (ssem0, ssem1)
        conds = []

        def scat_wait(k):
            bb = k % 2

            @pl.when(conds[k])
            def _():
                pltpu.make_async_copy(
                    bufs[bb], out_hbm.at[pl.ds(base + k * _R, _R)],
                    ssems[bb]).wait()

        for i in range(nsub):
            b = i % 2
            adjs = []
            cids = []
            for j in range(_R // 16):
                off2 = i * _R + j * 16
                cid16 = cid_v[pl.ds(off2, 16)]
                pos16 = lax.iota(jnp.int32, 16) + (gbase + off2)
                adj = jnp.clip(pos16 + cid16 * CHAIN_OFFSET, 0, MAX_LEN - 1)
                idx_v[i, pl.ds(j * 16, 16)] = adj
                adjs.append(adj)
                cids.append(cid16)
            cid_a = cids[0][0]
            cid_b = cids[-1][15]
            adj_a = adjs[0][0]
            adj_b = adjs[-1][15]
            is_linear = (cid_a == cid_b) & (adj_b == adj_a + (_R - 1))
            is_const = adj_b == adj_a
            res_mn = adj_a
            conds.append(jnp.logical_not(is_const))

            if i >= 2:
                scat_wait(i - 2)

            @pl.when(is_linear)
            def _():
                start = pl.multiple_of(res_mn, 8)
                pltpu.async_copy(pos_hbm.at[pl.ds(start, _R)], bufs[b],
                                 gsems[b]).wait()
                pltpu.async_copy(bufs[b], out_hbm.at[pl.ds(base + i * _R, _R)],
                                 ssems[b])

            @pl.when(jnp.logical_not(is_linear | is_const))
            def _():
                pltpu.async_copy(pos_hbm.at[idx_v.at[i]], bufs[b],
                                 gsems[b]).wait()
                pltpu.async_copy(bufs[b], out_hbm.at[pl.ds(base + i * _R, _R)],
                                 ssems[b])

        for k in (nsub - 2, nsub - 1):
            scat_wait(k)

    return _sc_gather


_BS = 512
_NB = SEQ_LEN // _BS
_NBH = _NB // 2


def _make_tc_body(half, aliased):
    base_blk = half * _NBH

    def body(*refs):
        if aliased:
            src_ref, prev_ref, x_ref, pos_ref, cid_ref, emb_ref, last_ref, o_ref = refs
        else:
            src_ref, x_ref, pos_ref, cid_ref, emb_ref, last_ref, o_ref = refs
        i = pl.program_id(0)
        cid = cid_ref[0, 0, :]
        n_chains = emb_ref.shape[0]
        onehot = (cid[:, None]
                  == lax.broadcasted_iota(jnp.int32, (_BS, n_chains), 1)
                  ).astype(jnp.float32)
        chain = jnp.dot(onehot, emb_ref[...],
                        preferred_element_type=jnp.float32,
                        precision=lax.Precision.HIGHEST)
        s = lax.broadcasted_iota(jnp.int32, (_BS, 1), 0) + (base_blk + i) * _BS
        clamped = (s + cid[:, None] * CHAIN_OFFSET) >= (MAX_LEN - 1)
        pos = jnp.where(clamped, last_ref[...], pos_ref[...])
        enc = pos + chain
        o_ref[...] = x_ref[...] + enc[None, :, :]

    return body


def _tc_add_half(src, x, pos_half, cid3, chain_embedding, pe_last, half,
                 y_prev=None):
    base_blk = half * _NBH
    aliased = y_prev is not None
    in_specs = [
        pl.BlockSpec((BATCH, _BS, D_MODEL),
                     lambda i, src: (0, base_blk + i, 0)),
        pl.BlockSpec((_BS, D_MODEL), lambda i, src: (src[i], 0)),
        pl.BlockSpec((1, 1, _BS), lambda i, src: (base_blk + i, 0, 0)),
        pl.BlockSpec(chain_embedding.shape, lambda i, src: (0, 0)),
        pl.BlockSpec((1, D_MODEL), lambda i, src: (0, 0)),
    ]
    operands = [src, x, pos_half, cid3, chain_embedding, pe_last]
    kwargs = {}
    if aliased:
        in_specs.insert(0, pl.BlockSpec((BATCH, _BS, D_MODEL),
                                        lambda i, src: (0, 0, 0)))
        operands.insert(1, y_prev)
        kwargs["input_output_aliases"] = {1: 0}
    grid_spec = pltpu.PrefetchScalarGridSpec(
        num_scalar_prefetch=1,
        grid=(_NBH,),
        in_specs=in_specs,
        out_specs=pl.BlockSpec((BATCH, _BS, D_MODEL),
                               lambda i, src: (0, base_blk + i, 0)),
    )
    return pl.pallas_call(
        _make_tc_body(half, aliased),
        grid_spec=grid_spec,
        out_shape=jax.ShapeDtypeStruct(x.shape, x.dtype),
        **kwargs,
    )(*operands)


def _half_src(cid, half):
    blk_ids = jnp.arange(_NBH, dtype=jnp.int32)
    gblk = blk_ids + half * _NBH
    first_cid = cid[half * _HALF:: _BS][:_NBH]
    unclamped = (gblk * _BS + first_cid * CHAIN_OFFSET) < (MAX_LEN - 1)
    return lax.cummax(jnp.where(unclamped, blk_ids, 0), axis=0)


def kernel(x, chain_id_tensor, pos_encoding, chain_embedding):
    cid = chain_id_tensor.astype(jnp.int32)
    cid3 = cid.reshape(_NB, 1, _BS)
    pe_last = pos_encoding[MAX_LEN - 1:, :]
    pos1 = _make_sc_gather(_HALF)(cid, pos_encoding)
    pos0 = _make_sc_gather(0)(cid, pos_encoding)
    y1 = _tc_add_half(_half_src(cid, 1), x, pos1, cid3, chain_embedding,
                      pe_last, half=1)
    return _tc_add_half(_half_src(cid, 0), x, pos0, cid3, chain_embedding,
                        pe_last, half=0, y_prev=y1)


# speedup vs baseline: 1.0364x; 1.0317x over previous
import functools

import jax
import jax.numpy as jnp
from jax import lax
from jax.experimental import pallas as pl
from jax.experimental.pallas import tpu as pltpu
from jax.experimental.pallas import tpu_sc as plsc

D_MODEL = 1024
MAX_LEN = 4096
CHAIN_OFFSET = 1000
SEQ_LEN = 4096
BATCH = 4

_R = 32


@functools.lru_cache(maxsize=1)
def _make_sc_gather():
    info = plsc.get_sparse_core_info()
    nc, ns = info.num_cores, info.num_subcores
    nw = nc * ns
    chunk = SEQ_LEN // nw
    nsub = chunk // _R
    mesh = plsc.VectorSubcoreMesh(core_axis_name="c", subcore_axis_name="s")

    @functools.partial(
        pl.kernel,
        mesh=mesh,
        out_type=jax.ShapeDtypeStruct((SEQ_LEN, D_MODEL), jnp.float32),
        scratch_types=[
            pltpu.VMEM((chunk,), jnp.int32),
            pltpu.VMEM((nsub, _R), jnp.int32),
            pltpu.VMEM((_R, D_MODEL), jnp.float32),
            pltpu.VMEM((_R, D_MODEL), jnp.float32),
            pltpu.VMEM((_R, D_MODEL), jnp.float32),
            pltpu.SemaphoreType.DMA,
            pltpu.SemaphoreType.DMA,
            pltpu.SemaphoreType.DMA,
            pltpu.SemaphoreType.DMA,
            pltpu.SemaphoreType.DMA,
            pltpu.SemaphoreType.DMA,
        ],
    )
    def _sc_gather(cid_hbm, pos_hbm, out_hbm, cid_v, idx_v, rows0, rows1,
                   rows2, gsem0, gsem1, gsem2, ssem0, ssem1, ssem2):
        wid = lax.axis_index("s") * nc + lax.axis_index("c")
        base = wid * chunk
        pltpu.sync_copy(cid_hbm.at[pl.ds(base, chunk)], cid_v)
        nbuf = 3
        bufs = (rows0, rows1, rows2)
        gsems = (gsem0, gsem1, gsem2)
        ssems = (ssem0, ssem1, ssem2)
        blocks = []

        for i in range(nsub):
            adjs = []
            cids = []
            for j in range(_R // 16):
                off = i * _R + j * 16
                cid16 = cid_v[pl.ds(off, 16)]
                pos16 = lax.iota(jnp.int32, 16) + (base + off)
                adj = jnp.clip(pos16 + cid16 * CHAIN_OFFSET, 0, MAX_LEN - 1)
                idx_v[i, pl.ds(j * 16, 16)] = adj
                adjs.append(adj)
                cids.append(cid16)
            cid_a = cids[0][0]
            cid_b = cids[-1][15]
            adj_a = adjs[0][0]
            adj_b = adjs[-1][15]
            is_linear = (cid_a == cid_b) & (adj_b == adj_a + (_R - 1))
            is_const = adj_b == adj_a
            blocks.append((jnp.logical_not(is_const), is_linear, adj_a))

        def g_issue(i):
            b = i % nbuf
            issued, linear, start = blocks[i]

            @pl.when(linear)
            def _():
                pltpu.async_copy(pos_hbm.at[pl.ds(pl.multiple_of(start, 8),
                                                  _R)], bufs[b], gsems[b])

            @pl.when(jnp.logical_not(linear) & issued)
            def _():
                pltpu.async_copy(pos_hbm.at[idx_v.at[i]], bufs[b], gsems[b])

        def g_wait(i):
            b = i % nbuf

            @pl.when(blocks[i][0])
            def _():
                pltpu.make_async_copy(pos_hbm.at[pl.ds(0, _R)], bufs[b],
                                      gsems[b]).wait()

        def s_issue(i):
            b = i % nbuf

            @pl.when(blocks[i][0])
            def _():
                pltpu.async_copy(bufs[b],
                                 out_hbm.at[pl.ds(base + i * _R, _R)],
                                 ssems[b])

        def s_wait(i):
            b = i % nbuf

            @pl.when(blocks[i][0])
            def _():
                pltpu.make_async_copy(bufs[b],
                                      out_hbm.at[pl.ds(base + i * _R, _R)],
                                      ssems[b]).wait()

        for i in range(min(nbuf, nsub)):
            g_issue(i)
        for i in range(nsub):
            g_wait(i)
            s_issue(i)
            if i + nbuf < nsub:
                s_wait(i)
                g_issue(i + nbuf)
        for i in range(max(0, nsub - nbuf), nsub):
            s_wait(i)

    return _sc_gather


_BS = 512
_NB = SEQ_LEN // _BS


def _tc_add_body(src_ref, x_ref, pos_ref, cid_ref, emb_ref, last_ref, o_ref):
    i = pl.program_id(0)
    cid = cid_ref[0, 0, :]
    n_chains = emb_ref.shape[0]
    onehot = (cid[:, None]
              == lax.broadcasted_iota(jnp.int32, (_BS, n_chains), 1)
              ).astype(jnp.float32)
    chain = jnp.dot(onehot, emb_ref[...], preferred_element_type=jnp.float32,
                    precision=lax.Precision.HIGHEST)
    s = lax.broadcasted_iota(jnp.int32, (_BS, 1), 0) + i * _BS
    clamped = (s + cid[:, None] * CHAIN_OFFSET) >= (MAX_LEN - 1)
    pos = jnp.where(clamped, last_ref[...], pos_ref[...])
    enc = pos + chain
    o_ref[...] = x_ref[...] + enc[None, :, :]


def _tc_add(src, x, pos_rows, cid3, chain_embedding, pe_last):
    grid_spec = pltpu.PrefetchScalarGridSpec(
        num_scalar_prefetch=1,
        grid=(_NB,),
        in_specs=[
            pl.BlockSpec((BATCH, _BS, D_MODEL), lambda i, src: (0, i, 0)),
            pl.BlockSpec((_BS, D_MODEL), lambda i, src: (src[i], 0)),
            pl.BlockSpec((1, 1, _BS), lambda i, src: (i, 0, 0)),
            pl.BlockSpec(chain_embedding.shape, lambda i, src: (0, 0)),
            pl.BlockSpec((1, D_MODEL), lambda i, src: (0, 0)),
        ],
        out_specs=pl.BlockSpec((BATCH, _BS, D_MODEL), lambda i, src: (0, i, 0)),
    )
    return pl.pallas_call(
        _tc_add_body,
        grid_spec=grid_spec,
        out_shape=jax.ShapeDtypeStruct(x.shape, x.dtype),
    )(src, x, pos_rows, cid3, chain_embedding, pe_last)


def kernel(x, chain_id_tensor, pos_encoding, chain_embedding):
    cid = chain_id_tensor.astype(jnp.int32)
    pos_rows = _make_sc_gather()(cid, pos_encoding)
    cid3 = cid.reshape(_NB, 1, _BS)
    pe_last = pos_encoding[MAX_LEN - 1:, :]
    blk_ids = jnp.arange(_NB, dtype=jnp.int32)
    first_cid = cid[:: _BS]
    unclamped = (blk_ids * _BS + first_cid * CHAIN_OFFSET) < (MAX_LEN - 1)
    src = lax.cummax(jnp.where(unclamped, blk_ids, 0), axis=0)
    return _tc_add(src, x, pos_rows, cid3, chain_embedding, pe_last)
